# Initial kernel scaffold; baseline (speedup 1.0000x reference)
import jax
import jax.numpy as jnp
from jax.experimental import pallas as pl


def kernel(h, item_neighbors_0_0, item_neighbors_1_0, item_neighbors_2_0, weight_neighbors_0_0, weight_neighbors_1_0, pos_neighbors_0_0, pos_neighbors_1_0, pos_before, pos_after, seq_hidden_local, mask_item, embedding, weight_list, bias_list, agg_W1, agg_W2, agg_W3, lin1_W, lin1_b, lin2_W, lin2_b, lin3_W):
    raise NotImplementedError("write your pallas kernel here")


# speedup vs baseline: 2.4024x; 2.4024x over previous
import functools

import jax
import jax.numpy as jnp
import numpy as np
from jax import lax
from jax.experimental import pallas as pl
from jax.experimental.pallas import tpu as pltpu
from jax.experimental.pallas import tpu_sc as plsc

B = 128
SEQ = 20
S = 10
CH = 4
CDIM = 32
INDIM = 128
P = 16
VOCAB = 100000
N1 = SEQ * S
N2 = SEQ * S * S
NTOT = N1 + N2
TOTAL = B * NTOT

ROWS_BLK = 2000


def _route_table_body(emb_ref, wl_ref, b_ref, ones_ref, out_ref):
    z = jnp.dot(emb_ref[...], wl_ref[...], preferred_element_type=jnp.float32)
    z = z + b_ref[...]
    nsq = jnp.dot(z * z, ones_ref[...], preferred_element_type=jnp.float32)
    out_ref[...] = z / jnp.maximum(jnp.sqrt(nsq), 1e-12)


def _route_table(emb, wlcat, bcat, blockones):
    grid = VOCAB // ROWS_BLK
    return pl.pallas_call(
        _route_table_body,
        grid=(grid,),
        in_specs=[
            pl.BlockSpec((ROWS_BLK, INDIM), lambda i: (i, 0)),
            pl.BlockSpec((INDIM, INDIM), lambda i: (0, 0)),
            pl.BlockSpec((1, INDIM), lambda i: (0, 0)),
            pl.BlockSpec((INDIM, INDIM), lambda i: (0, 0)),
        ],
        out_specs=pl.BlockSpec((ROWS_BLK, INDIM), lambda i: (i, 0)),
        out_shape=jax.ShapeDtypeStruct((VOCAB, INDIM), jnp.float32),
    )(emb, wlcat, bcat, blockones)


_NC = 2
_NS = 16
_NW = _NC * _NS
_PER_W = TOTAL // _NW
_CHUNK = 88
_NCHUNK = _PER_W // _CHUNK


def _sc_gather(table, idx_flat):
    mesh = plsc.VectorSubcoreMesh(core_axis_name="c", subcore_axis_name="s")

    @functools.partial(
        pl.kernel,
        mesh=mesh,
        out_type=jax.ShapeDtypeStruct((TOTAL, INDIM), jnp.float32),
        scratch_types=[
            pltpu.VMEM((_CHUNK,), jnp.int32),
            pltpu.VMEM((_CHUNK, INDIM), jnp.float32),
            pltpu.SemaphoreType.DMA,
        ],
    )
    def k(table_hbm, idx_hbm, out_hbm, idx_v, rows_v, sem):
        wid = lax.axis_index("s") * _NC + lax.axis_index("c")
        wbase = wid * _PER_W

        def body(c, carry):
            base = wbase + c * _CHUNK
            pltpu.sync_copy(idx_hbm.at[pl.ds(base, _CHUNK)], idx_v)
            pltpu.async_copy(table_hbm.at[idx_v], rows_v, sem).wait()
            pltpu.sync_copy(rows_v, out_hbm.at[pl.ds(base, _CHUNK)])
            return carry

        lax.fori_loop(0, _NCHUNK, body, 0)

    return k(table, idx_flat)


def _main_body(h_ref, shl_ref, mask_ref, g_ref, wp0_ref, wp1_ref,
               wlc_ref, bc_ref, bo_ref, pt_ref, e4_ref,
               w1c_ref, w17_ref, w2_ref, w3a_ref, w3b_ref,
               l1w_ref, l1b_ref, l2w_ref, l2b_ref, l3w_ref, out_ref):
    f32 = jnp.float32
    wlc = wlc_ref[...]
    bc = bc_ref[...]
    bo = bo_ref[...]

    def route(x):
        z = jnp.dot(x, wlc, preferred_element_type=f32) + bc
        nsq = jnp.dot(z * z, bo, preferred_element_type=f32)
        return z / jnp.maximum(jnp.sqrt(nsq), 1e-12)

    hb = h_ref[0]
    e0 = route(hb)
    item = route(shl_ref[0])
    msum = jnp.sum(mask_ref[0])
    srow = jnp.sum(item, axis=0, keepdims=True) / msum

    g = g_ref[0]
    g1 = g[0:N1]
    g2p = g[N1:NTOT]
    e4 = e4_ref[...]

    def agg(selfv, nbrp, wpp, hop, n):
        m = nbrp * srow
        a = (jnp.dot(m, w1c_ref[hop], preferred_element_type=f32)
             + jnp.dot(wpp, w17_ref[hop], preferred_element_type=f32))
        a = jnp.where(a >= 0, a, 0.2 * a)
        logit = jnp.dot(a, w2_ref[hop], preferred_element_type=f32)
        ls = [logit[k * n:(k + 1) * n] for k in range(S)]
        mx = ls[0]
        for k in range(1, S):
            mx = jnp.maximum(mx, ls[k])
        ex = [jnp.exp(ls[k] - mx) for k in range(S)]
        den = ex[0]
        for k in range(1, S):
            den = den + ex[k]
        pooled = jnp.zeros((n, INDIM), f32)
        for k in range(S):
            alpha = jnp.dot(ex[k] / den, e4, preferred_element_type=f32)
            pooled = pooled + alpha * nbrp[k * n:(k + 1) * n]
        o = (jnp.dot(selfv, w3a_ref[hop], preferred_element_type=f32)
             + jnp.dot(pooled, w3b_ref[hop], preferred_element_type=f32))
        return jnp.maximum(o, 0.0)

    h1 = agg(g1, g2p, wp1_ref[0], 0, N1)
    pt = pt_ref[...]
    g1p = jnp.dot(pt, g1, preferred_element_type=f32)
    h0 = agg(e0, g1p, wp0_ref[0], 0, SEQ)
    h1p = jnp.dot(pt, h1, preferred_element_type=f32)
    fin = agg(h0, h1p, wp0_ref[0], 1, SEQ)

    q = (jnp.dot(fin, l1w_ref[...], preferred_element_type=f32) + l1b_ref[...]
         + jnp.dot(hb, l2w_ref[...], preferred_element_type=f32) + l2b_ref[...])
    alpha = jnp.dot(jax.nn.sigmoid(q), l3w_ref[...], preferred_element_type=f32)
    out_ref[0] = alpha * hb + (1.0 - alpha) * fin


def _main(h, shl, mask3, g, wp0, wp1, wlcat, bcat, blockones, pt, e4,
          bdw1c, w17, bdw2, bdw3a, bdw3b, l1w, l1b, l2w, l2b, l3w):
    full = lambda shape: pl.BlockSpec(shape, lambda b: (0,) * len(shape))
    batch = lambda shape: pl.BlockSpec((1,) + shape, lambda b: (b,) + (0,) * len(shape))
    return pl.pallas_call(
        _main_body,
        grid=(B,),
        in_specs=[
            batch((SEQ, INDIM)),
            batch((SEQ, INDIM)),
            batch((1, SEQ)),
            batch((NTOT, INDIM)),
            batch((N1, P + 1)),
            batch((N2, P + 1)),
            full((INDIM, INDIM)),
            full((1, INDIM)),
            full((INDIM, INDIM)),
            full((N1, N1)),
            full((CH, INDIM)),
            full((2, INDIM, INDIM)),
            full((2, P + 1, INDIM)),
            full((2, INDIM, CH)),
            full((2, INDIM, INDIM)),
            full((2, INDIM, INDIM)),
            full((INDIM, INDIM)),
            full((1, INDIM)),
            full((INDIM, INDIM)),
            full((1, INDIM)),
            full((INDIM, 1)),
        ],
        out_specs=pl.BlockSpec((1, SEQ, INDIM), lambda b: (b, 0, 0)),
        out_shape=jax.ShapeDtypeStruct((B, SEQ, INDIM), jnp.float32),
    )(h, shl, mask3, g, wp0, wp1, wlcat, bcat, blockones, pt, e4,
      bdw1c, w17, bdw2, bdw3a, bdw3b, l1w, l1b, l2w, l2b, l3w)


_PT_NP = np.zeros((N1, N1), np.float32)
_r = np.arange(N1)
_PT_NP[_r, (_r % SEQ) * S + (_r // SEQ)] = 1.0


def kernel(h, item_neighbors_0_0, item_neighbors_1_0, item_neighbors_2_0,
           weight_neighbors_0_0, weight_neighbors_1_0, pos_neighbors_0_0,
           pos_neighbors_1_0, pos_before, pos_after, seq_hidden_local,
           mask_item, embedding, weight_list, bias_list, agg_W1, agg_W2,
           agg_W3, lin1_W, lin1_b, lin2_W, lin2_b, lin3_W):
    f32 = jnp.float32
    eye4 = jnp.eye(CH, dtype=f32)

    wlcat = jnp.concatenate([weight_list[j] for j in range(CH)], axis=1)
    bcat = jnp.concatenate([bias_list[j] for j in range(CH)], axis=1)
    blockones = jnp.kron(eye4, jnp.ones((CDIM, CDIM), f32))
    e4 = jnp.kron(eye4, jnp.ones((1, CDIM), f32))
    bdw1c = jnp.stack([jnp.kron(eye4, agg_W1[i, :CDIM]) for i in range(2)])
    w17 = jnp.stack([jnp.tile(agg_W1[i, CDIM:], (1, CH)) for i in range(2)])
    bdw2 = jnp.stack([jnp.kron(eye4, agg_W2[i]) for i in range(2)])
    bdw3a = jnp.stack([jnp.kron(eye4, agg_W3[i, :CDIM]) for i in range(2)])
    bdw3b = jnp.stack([jnp.kron(eye4, agg_W3[i, CDIM:]) for i in range(2)])
    pt = jnp.asarray(_PT_NP)

    i1 = item_neighbors_1_0.astype(jnp.int32)
    i2p = (item_neighbors_2_0.astype(jnp.int32)
           .reshape(B, N1, S).transpose(0, 2, 1).reshape(B, N2))
    idx_flat = jnp.concatenate([i1, i2p], axis=1).reshape(TOTAL)

    wp0 = jnp.concatenate(
        [weight_neighbors_0_0.reshape(B, SEQ, S, 1),
         pos_neighbors_0_0], axis=-1)
    wp0 = wp0.transpose(0, 2, 1, 3).reshape(B, N1, P + 1)
    wp1 = jnp.concatenate(
        [weight_neighbors_1_0.reshape(B, N1, S, 1),
         pos_neighbors_1_0], axis=-1)
    wp1 = wp1.transpose(0, 2, 1, 3).reshape(B, N2, P + 1)

    mask3 = mask_item.reshape(B, 1, SEQ)
    l1b = lin1_b.reshape(1, INDIM)
    l2b = lin2_b.reshape(1, INDIM)

    table = _route_table(embedding, wlcat, bcat, blockones)
    g = _sc_gather(table, idx_flat).reshape(B, NTOT, INDIM)
    return _main(h, seq_hidden_local, mask3, g, wp0, wp1,
                 wlcat, bcat, blockones, pt, e4,
                 bdw1c, w17, bdw2, bdw3a, bdw3b,
                 lin1_W, l1b, lin2_W, l2b, lin3_W)
